# baseline (speedup 1.0000x reference)
import functools

import jax
import jax.numpy as jnp
from jax import lax
from jax.experimental import pallas as pl
from jax.experimental.pallas import tpu as pltpu
from jax.experimental.pallas import tpu_sc as plsc

ROWS = 16384
COLS = 1000
L = 16
NC = 2
NS = 16
NW = NC * NS
G = 16
GC = G * COLS
NCH = COLS // L
TAIL = COLS - NCH * L

SC_ROWS = 8192
TC_ROWS = ROWS - SC_ROWS
TC_BLK = 2048

RPW = SC_ROWS // NW
NG = RPW // G

_mesh = plsc.VectorSubcoreMesh(core_axis_name="c", subcore_axis_name="s")


@functools.partial(
    pl.kernel,
    mesh=_mesh,
    compiler_params=pltpu.CompilerParams(needs_layout_passes=False),
    out_type=jax.ShapeDtypeStruct((NW, RPW * L), jnp.float32),
    scratch_types=[
        pltpu.VMEM((GC + L,), jnp.float32),
        pltpu.VMEM((GC + L,), jnp.float32),
        pltpu.VMEM((RPW,), jnp.int32),
        pltpu.VMEM((RPW * L,), jnp.float32),
        pltpu.SemaphoreType.DMA,
        pltpu.SemaphoreType.DMA,
    ],
)
def _sc_row_sums(pred_hbm, idx_hbm, out_hbm, buf_a, buf_b, tall, sums,
                 sem_a, sem_b):
    wid = lax.axis_index("s") * NC + lax.axis_index("c")
    base_row = wid * RPW
    lane = lax.iota(jnp.int32, L)

    pltpu.sync_copy(idx_hbm.at[pl.ds(base_row, RPW)], tall)

    def start_fetch(g, buf, sem):
        src = pred_hbm.at[pl.ds((base_row + g * G) * COLS, GC)]
        pltpu.make_async_copy(src, buf.at[pl.ds(0, GC)], sem).start()

    def wait_fetch(buf, sem):
        src = pred_hbm.at[pl.ds(0, GC)]
        pltpu.make_async_copy(src, buf.at[pl.ds(0, GC)], sem).wait()

    def process_group(g, buf):
        def row_body(r, carry):
            t_b = plsc.load_gather(tall, [jnp.full((L,), g * G + r, jnp.int32)])
            roff = r * COLS
            s_b = plsc.load_gather(buf, [t_b + roff])
            bits = plsc.bitcast(s_b, jnp.int32)
            up = jnp.where(s_b > 0.0, bits + 1,
                           jnp.where(s_b < 0.0, bits - 1,
                                     jnp.int32(0x00800000)))
            thr_hi = plsc.bitcast(up, jnp.float32)
            u = t_b - lane

            acc = jnp.zeros((L,), jnp.float32)
            for c in range(NCH):
                v = buf[pl.ds(roff + c * L, L)]
                thr = jnp.where(c * L >= u, thr_hi, s_b)
                acc = acc + jnp.where(v < thr, jnp.exp(v - s_b), 0.0)
            v = buf[pl.ds(roff + NCH * L, L)]
            thr = jnp.where(NCH * L >= u, thr_hi, s_b)
            m = (v < thr) & (lane < TAIL)
            acc = acc + jnp.where(m, jnp.exp(v - s_b), 0.0)

            sums[pl.ds((g * G + r) * L, L)] = acc
            return carry

        lax.fori_loop(0, G, row_body, 0)

    start_fetch(0, buf_a, sem_a)

    def pair_body(i, carry):
        g0 = i * 2
        wait_fetch(buf_a, sem_a)
        start_fetch(g0 + 1, buf_b, sem_b)
        process_group(g0, buf_a)
        wait_fetch(buf_b, sem_b)

        @pl.when(i < NG // 2 - 1)
        def _():
            start_fetch(g0 + 2, buf_a, sem_a)

        process_group(g0 + 1, buf_b)
        return carry

    lax.fori_loop(0, NG // 2, pair_body, 0)
    pltpu.sync_copy(sums, out_hbm.at[wid])


def _tc_body(x_ref, t_ref, o_ref):
    x = x_ref[...]
    t2 = t_ref[0, 0, :].reshape(TC_BLK, 1)
    cols = lax.broadcasted_iota(jnp.int32, (TC_BLK, COLS), 1)
    s_t = jnp.sum(jnp.where(cols == t2, x, 0.0), axis=1, keepdims=True)
    d = x - s_t
    dthr = jnp.where(cols >= t2, 1e-30, 0.0)
    rs = jnp.sum(jnp.where(d < dthr, jnp.exp(d), 0.0), axis=1)
    part = jnp.sum(jnp.log(rs))

    @pl.when(pl.program_id(0) == 0)
    def _():
        o_ref[0, 0] = 0.0

    o_ref[0, 0] += part


_tc_partial = pl.pallas_call(
    _tc_body,
    grid=(TC_ROWS // TC_BLK,),
    in_specs=[
        pl.BlockSpec((TC_BLK, COLS), lambda i: (i, 0)),
        pl.BlockSpec((1, 1, TC_BLK), lambda i: (i, 0, 0)),
    ],
    out_specs=pl.BlockSpec(memory_space=pltpu.SMEM),
    out_shape=jax.ShapeDtypeStruct((1, 1), jnp.float32),
)


def _merge_body(s_ref, p_ref, o_ref):
    row_sums = jnp.sum(s_ref[...], axis=1)
    o_ref[0, 0] = (jnp.sum(jnp.log(row_sums)) + p_ref[0, 0]) / ROWS


_merge = pl.pallas_call(
    _merge_body,
    in_specs=[
        pl.BlockSpec((SC_ROWS, L), lambda: (0, 0)),
        pl.BlockSpec(memory_space=pltpu.SMEM),
    ],
    out_specs=pl.BlockSpec(memory_space=pltpu.SMEM),
    out_shape=jax.ShapeDtypeStruct((1, 1), jnp.float32),
)


@jax.jit
def kernel(pred_scores, true_indices):
    if pred_scores.ndim == 1:
        pred_scores = pred_scores[None, :]
    ti = true_indices.reshape(-1).astype(jnp.int32)
    ps_sc = pred_scores[:SC_ROWS].reshape(-1)
    ti_sc = ti[:SC_ROWS]
    ps_tc = pred_scores[SC_ROWS:]
    ti_tc = ti[SC_ROWS:].reshape(TC_ROWS // TC_BLK, 1, TC_BLK)
    sc_sums = _sc_row_sums(ps_sc, ti_sc)
    tc_part = _tc_partial(ps_tc, ti_tc)
    return _merge(sc_sums.reshape(SC_ROWS, L), tc_part)[0, 0]


# speedup vs baseline: 1.1219x; 1.1219x over previous
import functools

import jax
import jax.numpy as jnp
from jax import lax
from jax.experimental import pallas as pl
from jax.experimental.pallas import tpu as pltpu
from jax.experimental.pallas import tpu_sc as plsc

ROWS = 16384
COLS = 1000
L = 16
NC = 2
NS = 16
NW = NC * NS
G = 16
GC = G * COLS
NCH = COLS // L
TAIL = COLS - NCH * L

SC_ROWS = 2048
TC_ROWS = ROWS - SC_ROWS
TC_BLK = 2048

RPW = SC_ROWS // NW
NG = RPW // G

_mesh = plsc.VectorSubcoreMesh(core_axis_name="c", subcore_axis_name="s")


@functools.partial(
    pl.kernel,
    mesh=_mesh,
    compiler_params=pltpu.CompilerParams(needs_layout_passes=False),
    out_type=jax.ShapeDtypeStruct((NW, RPW * L), jnp.float32),
    scratch_types=[
        pltpu.VMEM((GC + L,), jnp.float32),
        pltpu.VMEM((GC + L,), jnp.float32),
        pltpu.VMEM((RPW,), jnp.int32),
        pltpu.VMEM((RPW * L,), jnp.float32),
        pltpu.SemaphoreType.DMA,
        pltpu.SemaphoreType.DMA,
    ],
)
def _sc_row_sums(pred_hbm, idx_hbm, out_hbm, buf_a, buf_b, tall, sums,
                 sem_a, sem_b):
    wid = lax.axis_index("s") * NC + lax.axis_index("c")
    base_row = wid * RPW
    lane = lax.iota(jnp.int32, L)

    pltpu.sync_copy(idx_hbm.at[pl.ds(base_row, RPW)], tall)

    def start_fetch(g, buf, sem):
        src = pred_hbm.at[pl.ds((base_row + g * G) * COLS, GC)]
        pltpu.make_async_copy(src, buf.at[pl.ds(0, GC)], sem).start()

    def wait_fetch(buf, sem):
        src = pred_hbm.at[pl.ds(0, GC)]
        pltpu.make_async_copy(src, buf.at[pl.ds(0, GC)], sem).wait()

    def process_group(g, buf):
        def row_body(r, carry):
            t_b = plsc.load_gather(tall, [jnp.full((L,), g * G + r, jnp.int32)])
            roff = r * COLS
            s_b = plsc.load_gather(buf, [t_b + roff])
            bits = plsc.bitcast(s_b, jnp.int32)
            up = jnp.where(s_b > 0.0, bits + 1,
                           jnp.where(s_b < 0.0, bits - 1,
                                     jnp.int32(0x00800000)))
            thr_hi = plsc.bitcast(up, jnp.float32)
            u = t_b - lane

            acc = jnp.zeros((L,), jnp.float32)
            for c in range(NCH):
                v = buf[pl.ds(roff + c * L, L)]
                thr = jnp.where(c * L >= u, thr_hi, s_b)
                acc = acc + jnp.where(v < thr, jnp.exp(v - s_b), 0.0)
            v = buf[pl.ds(roff + NCH * L, L)]
            thr = jnp.where(NCH * L >= u, thr_hi, s_b)
            m = (v < thr) & (lane < TAIL)
            acc = acc + jnp.where(m, jnp.exp(v - s_b), 0.0)

            sums[pl.ds((g * G + r) * L, L)] = acc
            return carry

        lax.fori_loop(0, G, row_body, 0)

    start_fetch(0, buf_a, sem_a)

    def pair_body(i, carry):
        g0 = i * 2
        wait_fetch(buf_a, sem_a)
        start_fetch(g0 + 1, buf_b, sem_b)
        process_group(g0, buf_a)
        wait_fetch(buf_b, sem_b)

        @pl.when(i < NG // 2 - 1)
        def _():
            start_fetch(g0 + 2, buf_a, sem_a)

        process_group(g0 + 1, buf_b)
        return carry

    lax.fori_loop(0, NG // 2, pair_body, 0)
    pltpu.sync_copy(sums, out_hbm.at[wid])


def _tc_body(x_ref, t_ref, o_ref):
    x = x_ref[...]
    t2 = t_ref[0, 0, :].reshape(TC_BLK, 1)
    cols = lax.broadcasted_iota(jnp.int32, (TC_BLK, COLS), 1)
    s_t = jnp.sum(jnp.where(cols == t2, x, 0.0), axis=1, keepdims=True)
    d = x - s_t
    dthr = jnp.where(cols >= t2, 1e-30, 0.0)
    rs = jnp.sum(jnp.where(d < dthr, jnp.exp(d), 0.0), axis=1)
    part = jnp.sum(jnp.log(rs))

    @pl.when(pl.program_id(0) == 0)
    def _():
        o_ref[0, 0] = 0.0

    o_ref[0, 0] += part


_tc_partial = pl.pallas_call(
    _tc_body,
    grid=(TC_ROWS // TC_BLK,),
    in_specs=[
        pl.BlockSpec((TC_BLK, COLS), lambda i: (i, 0)),
        pl.BlockSpec((1, 1, TC_BLK), lambda i: (i, 0, 0)),
    ],
    out_specs=pl.BlockSpec(memory_space=pltpu.SMEM),
    out_shape=jax.ShapeDtypeStruct((1, 1), jnp.float32),
)


def _merge_body(s_ref, p_ref, o_ref):
    row_sums = jnp.sum(s_ref[...], axis=1)
    o_ref[0, 0] = (jnp.sum(jnp.log(row_sums)) + p_ref[0, 0]) / ROWS


_merge = pl.pallas_call(
    _merge_body,
    in_specs=[
        pl.BlockSpec((SC_ROWS, L), lambda: (0, 0)),
        pl.BlockSpec(memory_space=pltpu.SMEM),
    ],
    out_specs=pl.BlockSpec(memory_space=pltpu.SMEM),
    out_shape=jax.ShapeDtypeStruct((1, 1), jnp.float32),
)


@jax.jit
def kernel(pred_scores, true_indices):
    if pred_scores.ndim == 1:
        pred_scores = pred_scores[None, :]
    ti = true_indices.reshape(-1).astype(jnp.int32)
    ps_sc = pred_scores[:SC_ROWS].reshape(-1)
    ti_sc = ti[:SC_ROWS]
    ps_tc = pred_scores[SC_ROWS:]
    ti_tc = ti[SC_ROWS:].reshape(TC_ROWS // TC_BLK, 1, TC_BLK)
    sc_sums = _sc_row_sums(ps_sc, ti_sc)
    tc_part = _tc_partial(ps_tc, ti_tc)
    return _merge(sc_sums.reshape(SC_ROWS, L), tc_part)[0, 0]


# speedup vs baseline: 1.3944x; 1.2428x over previous
import functools

import jax
import jax.numpy as jnp
from jax import lax
from jax.experimental import pallas as pl
from jax.experimental.pallas import tpu as pltpu
from jax.experimental.pallas import tpu_sc as plsc

ROWS = 16384
COLS = 1000
L = 16
NC = 2
NS = 16
NW = NC * NS
G = 16
GC = G * COLS
NCH = COLS // L
TAIL = COLS - NCH * L

SC_ROWS = 2048
TC_ROWS = ROWS - SC_ROWS
TC_BLK = 2048

RPW = SC_ROWS // NW
NG = RPW // G

_mesh = plsc.VectorSubcoreMesh(core_axis_name="c", subcore_axis_name="s")


@functools.partial(
    pl.kernel,
    mesh=_mesh,
    compiler_params=pltpu.CompilerParams(needs_layout_passes=False),
    out_type=jax.ShapeDtypeStruct((NW, RPW * L), jnp.float32),
    scratch_types=[
        pltpu.VMEM((GC + L,), jnp.float32),
        pltpu.VMEM((GC + L,), jnp.float32),
        pltpu.VMEM((RPW,), jnp.int32),
        pltpu.VMEM((RPW * L,), jnp.float32),
        pltpu.SemaphoreType.DMA,
        pltpu.SemaphoreType.DMA,
    ],
)
def _sc_row_sums(pred_hbm, idx_hbm, out_hbm, buf_a, buf_b, tall, sums,
                 sem_a, sem_b):
    wid = lax.axis_index("s") * NC + lax.axis_index("c")
    base_row = wid * RPW
    lane = lax.iota(jnp.int32, L)

    pltpu.sync_copy(idx_hbm.at[pl.ds(base_row, RPW)], tall)

    def start_fetch(g, buf, sem):
        src = pred_hbm.at[pl.ds((base_row + g * G) * COLS, GC)]
        pltpu.make_async_copy(src, buf.at[pl.ds(0, GC)], sem).start()

    def wait_fetch(buf, sem):
        src = pred_hbm.at[pl.ds(0, GC)]
        pltpu.make_async_copy(src, buf.at[pl.ds(0, GC)], sem).wait()

    def process_group(g, buf):
        def row_body(r, carry):
            t_b = plsc.load_gather(tall, [jnp.full((L,), g * G + r, jnp.int32)])
            roff = r * COLS
            s_b = plsc.load_gather(buf, [t_b + roff])
            bits = plsc.bitcast(s_b, jnp.int32)
            up = jnp.where(s_b > 0.0, bits + 1,
                           jnp.where(s_b < 0.0, bits - 1,
                                     jnp.int32(0x00800000)))
            thr_hi = plsc.bitcast(up, jnp.float32)
            u = t_b - lane

            acc = jnp.zeros((L,), jnp.float32)
            for c in range(NCH):
                v = buf[pl.ds(roff + c * L, L)]
                thr = jnp.where(c * L >= u, thr_hi, s_b)
                acc = acc + jnp.where(v < thr, jnp.exp(v - s_b), 0.0)
            v = buf[pl.ds(roff + NCH * L, L)]
            thr = jnp.where(NCH * L >= u, thr_hi, s_b)
            m = (v < thr) & (lane < TAIL)
            acc = acc + jnp.where(m, jnp.exp(v - s_b), 0.0)

            sums[pl.ds((g * G + r) * L, L)] = acc
            return carry

        lax.fori_loop(0, G, row_body, 0)

    start_fetch(0, buf_a, sem_a)

    def pair_body(i, carry):
        g0 = i * 2
        wait_fetch(buf_a, sem_a)
        start_fetch(g0 + 1, buf_b, sem_b)
        process_group(g0, buf_a)
        wait_fetch(buf_b, sem_b)

        @pl.when(i < NG // 2 - 1)
        def _():
            start_fetch(g0 + 2, buf_a, sem_a)

        process_group(g0 + 1, buf_b)
        return carry

    lax.fori_loop(0, NG // 2, pair_body, 0)
    pltpu.sync_copy(sums, out_hbm.at[wid])


def _tc_body(x_ref, t_ref, o_ref):
    x = x_ref[...]
    t2 = t_ref[0, 0, :].reshape(TC_BLK, 1)
    cols = lax.broadcasted_iota(jnp.int32, (TC_BLK, COLS), 1)
    s_t = jnp.sum(jnp.where(cols == t2, x, 0.0), axis=1, keepdims=True)
    d = x - s_t
    dthr = jnp.where(cols >= t2, 1e-30, 0.0)
    rs = jnp.sum(jnp.where(d < dthr, jnp.exp(d), 0.0), axis=1)
    part = jnp.sum(jnp.log(rs))

    @pl.when(pl.program_id(0) == 0)
    def _():
        o_ref[0, 0] = 0.0

    o_ref[0, 0] += part


_TC_OFF = SC_ROWS // TC_BLK

_tc_partial = pl.pallas_call(
    _tc_body,
    grid=(TC_ROWS // TC_BLK,),
    in_specs=[
        pl.BlockSpec((TC_BLK, COLS), lambda i: (i + _TC_OFF, 0)),
        pl.BlockSpec((1, 1, TC_BLK), lambda i: (i + _TC_OFF, 0, 0)),
    ],
    out_specs=pl.BlockSpec(memory_space=pltpu.SMEM),
    out_shape=jax.ShapeDtypeStruct((1, 1), jnp.float32),
)


def _merge_body(s_ref, p_ref, o_ref):
    row_sums = jnp.sum(s_ref[...], axis=1)
    o_ref[0, 0] = (jnp.sum(jnp.log(row_sums)) + p_ref[0, 0]) / ROWS


_merge = pl.pallas_call(
    _merge_body,
    in_specs=[
        pl.BlockSpec((SC_ROWS, L), lambda: (0, 0)),
        pl.BlockSpec(memory_space=pltpu.SMEM),
    ],
    out_specs=pl.BlockSpec(memory_space=pltpu.SMEM),
    out_shape=jax.ShapeDtypeStruct((1, 1), jnp.float32),
)


@jax.jit
def kernel(pred_scores, true_indices):
    if pred_scores.ndim == 1:
        pred_scores = pred_scores[None, :]
    ti = true_indices.reshape(-1).astype(jnp.int32)
    ps_sc = pred_scores[:SC_ROWS].reshape(-1)
    ti_sc = ti[:SC_ROWS]
    ti_3d = ti.reshape(ROWS // TC_BLK, 1, TC_BLK)
    sc_sums = _sc_row_sums(ps_sc, ti_sc)
    tc_part = _tc_partial(pred_scores, ti_3d)
    return _merge(sc_sums.reshape(SC_ROWS, L), tc_part)[0, 0]


# speedup vs baseline: 1.5062x; 1.0802x over previous
import functools

import jax
import jax.numpy as jnp
from jax import lax
from jax.experimental import pallas as pl
from jax.experimental.pallas import tpu as pltpu
from jax.experimental.pallas import tpu_sc as plsc

ROWS = 16384
COLS = 1000
L = 16
NC = 2
NS = 16
NW = NC * NS
G = 16
GC = G * COLS
NCH = COLS // L
TAIL = COLS - NCH * L

SC_ROWS = 1024
TC_ROWS = ROWS - SC_ROWS
TC_BLK = 1024

RPW = SC_ROWS // NW
NG = RPW // G

_mesh = plsc.VectorSubcoreMesh(core_axis_name="c", subcore_axis_name="s")


@functools.partial(
    pl.kernel,
    mesh=_mesh,
    compiler_params=pltpu.CompilerParams(needs_layout_passes=False),
    out_type=jax.ShapeDtypeStruct((NW, RPW * L), jnp.float32),
    scratch_types=[
        pltpu.VMEM((GC + L,), jnp.float32),
        pltpu.VMEM((GC + L,), jnp.float32),
        pltpu.VMEM((RPW,), jnp.int32),
        pltpu.VMEM((RPW * L,), jnp.float32),
        pltpu.SemaphoreType.DMA,
        pltpu.SemaphoreType.DMA,
    ],
)
def _sc_row_sums(pred_hbm, idx_hbm, out_hbm, buf_a, buf_b, tall, sums,
                 sem_a, sem_b):
    wid = lax.axis_index("s") * NC + lax.axis_index("c")
    base_row = wid * RPW
    lane = lax.iota(jnp.int32, L)

    pltpu.sync_copy(idx_hbm.at[pl.ds(base_row, RPW)], tall)

    def start_fetch(g, buf, sem):
        src = pred_hbm.at[pl.ds((base_row + g * G) * COLS, GC)]
        pltpu.make_async_copy(src, buf.at[pl.ds(0, GC)], sem).start()

    def wait_fetch(buf, sem):
        src = pred_hbm.at[pl.ds(0, GC)]
        pltpu.make_async_copy(src, buf.at[pl.ds(0, GC)], sem).wait()

    def process_group(g, buf):
        def row_body(r, carry):
            t_b = plsc.load_gather(tall, [jnp.full((L,), g * G + r, jnp.int32)])
            roff = r * COLS
            s_b = plsc.load_gather(buf, [t_b + roff])
            bits = plsc.bitcast(s_b, jnp.int32)
            up = jnp.where(s_b > 0.0, bits + 1,
                           jnp.where(s_b < 0.0, bits - 1,
                                     jnp.int32(0x00800000)))
            thr_hi = plsc.bitcast(up, jnp.float32)
            u = t_b - lane

            acc = jnp.zeros((L,), jnp.float32)
            for c in range(NCH):
                v = buf[pl.ds(roff + c * L, L)]
                thr = jnp.where(c * L >= u, thr_hi, s_b)
                acc = acc + jnp.where(v < thr, jnp.exp(v - s_b), 0.0)
            v = buf[pl.ds(roff + NCH * L, L)]
            thr = jnp.where(NCH * L >= u, thr_hi, s_b)
            m = (v < thr) & (lane < TAIL)
            acc = acc + jnp.where(m, jnp.exp(v - s_b), 0.0)

            sums[pl.ds((g * G + r) * L, L)] = acc
            return carry

        lax.fori_loop(0, G, row_body, 0)

    start_fetch(0, buf_a, sem_a)

    def pair_body(i, carry):
        g0 = i * 2
        wait_fetch(buf_a, sem_a)
        start_fetch(g0 + 1, buf_b, sem_b)
        process_group(g0, buf_a)
        wait_fetch(buf_b, sem_b)

        @pl.when(i < NG // 2 - 1)
        def _():
            start_fetch(g0 + 2, buf_a, sem_a)

        process_group(g0 + 1, buf_b)
        return carry

    lax.fori_loop(0, NG // 2, pair_body, 0)
    pltpu.sync_copy(sums, out_hbm.at[wid])


def _tc_body(x_ref, t_ref, o_ref):
    x = x_ref[...]
    t2 = t_ref[0, 0, :].reshape(TC_BLK, 1)
    cols = lax.broadcasted_iota(jnp.int32, (TC_BLK, COLS), 1)
    s_t = jnp.sum(jnp.where(cols == t2, x, 0.0), axis=1, keepdims=True)
    d = x - s_t
    dthr = jnp.where(cols >= t2, 1e-30, 0.0)
    rs = jnp.sum(jnp.where(d < dthr, jnp.exp(d), 0.0), axis=1)
    part = jnp.sum(jnp.log(rs))

    @pl.when(pl.program_id(0) == 0)
    def _():
        o_ref[0, 0] = 0.0

    o_ref[0, 0] += part


_TC_OFF = SC_ROWS // TC_BLK

_tc_partial = pl.pallas_call(
    _tc_body,
    grid=(TC_ROWS // TC_BLK,),
    in_specs=[
        pl.BlockSpec((TC_BLK, COLS), lambda i: (i + _TC_OFF, 0)),
        pl.BlockSpec((1, 1, TC_BLK), lambda i: (i + _TC_OFF, 0, 0)),
    ],
    out_specs=pl.BlockSpec(memory_space=pltpu.SMEM),
    out_shape=jax.ShapeDtypeStruct((1, 1), jnp.float32),
)


def _merge_body(s_ref, p_ref, o_ref):
    row_sums = jnp.sum(s_ref[...], axis=1)
    o_ref[0, 0] = (jnp.sum(jnp.log(row_sums)) + p_ref[0, 0]) / ROWS


_merge = pl.pallas_call(
    _merge_body,
    in_specs=[
        pl.BlockSpec((SC_ROWS, L), lambda: (0, 0)),
        pl.BlockSpec(memory_space=pltpu.SMEM),
    ],
    out_specs=pl.BlockSpec(memory_space=pltpu.SMEM),
    out_shape=jax.ShapeDtypeStruct((1, 1), jnp.float32),
)


@jax.jit
def kernel(pred_scores, true_indices):
    if pred_scores.ndim == 1:
        pred_scores = pred_scores[None, :]
    ti = true_indices.reshape(-1).astype(jnp.int32)
    ps_sc = pred_scores[:SC_ROWS].reshape(-1)
    ti_sc = ti[:SC_ROWS]
    ti_3d = ti.reshape(ROWS // TC_BLK, 1, TC_BLK)
    sc_sums = _sc_row_sums(ps_sc, ti_sc)
    tc_part = _tc_partial(pred_scores, ti_3d)
    return _merge(sc_sums.reshape(SC_ROWS, L), tc_part)[0, 0]
